# baseline (speedup 1.0000x reference)
import functools

import jax
import jax.numpy as jnp
from jax import lax
from jax.experimental import pallas as pl
from jax.experimental.pallas import tpu as pltpu
from jax.experimental.pallas import tpu_sc as plsc

N = 10000
E = 320000
D = 128
H = 128
L = 3
C = 10
G = 64

NC = 2
NS = 16
NW = NC * NS
EPW = E // NW
CH = 40
CB = EPW // CH
NPH = 5
CPP = CB // NPH
ND = 4
NP = 10240
RPS = NP // NS


def _sc_agg_body(x_hbm, ei_hbm, zeros_hbm, out_hbm,
                 idx_v, rows_v, sem0, sem1, sem2, sem3, agg_sh):
    c = lax.axis_index("c")
    s = lax.axis_index("s")
    w = s * NC + c
    sems = (sem0, sem1, sem2, sem3)

    def load_phase(ph, parity):
        pltpu.sync_copy(ei_hbm.at[w].at[ph], idx_v.at[parity])

    def gather(parity, cc, buf, sm):
        pltpu.async_copy(x_hbm.at[idx_v.at[parity].at[0].at[cc]],
                         rows_v.at[buf], sm)

    def wait_one(sm):
        pltpu.make_async_copy(x_hbm.at[idx_v.at[0].at[0].at[0]],
                              rows_v.at[0], sm).wait()

    def scatter(parity, cc, buf):
        pltpu.sync_copy(rows_v.at[buf], agg_sh.at[idx_v.at[parity].at[1].at[cc]],
                        add=True)

    load_phase(0, 0)
    for j in range(ND):
        gather(0, j, j, sems[j])
    base = s * RPS
    pltpu.sync_copy(zeros_hbm, agg_sh.at[pl.ds(base, 8)])
    have = 8
    while have < RPS:
        n = min(have, RPS - have)
        pltpu.sync_copy(agg_sh.at[pl.ds(base, n)],
                        agg_sh.at[pl.ds(base + have, n)])
        have += n
    plsc.subcore_barrier()

    for ph in range(NPH):
        pb = ph & 1
        if ph + 1 < NPH:
            load_phase(ph + 1, (ph + 1) & 1)
        rs = [(2 * ph + j) % ND for j in range(ND)]

        def inner(v, carry, pb=pb, rs=rs):
            cc = ND * v
            for j in range(ND):
                r = rs[j]
                wait_one(sems[r])
                scatter(pb, cc + j, r)
                gather(pb, cc + j + ND, r, sems[r])
            return carry

        lax.fori_loop(0, (CPP - 2 * ND) // ND + 1, inner, 0)
        for cc in range(CPP - 2 - ND, CPP):
            r = (2 * ph + cc) % ND
            wait_one(sems[r])
            scatter(pb, cc, r)
            nxt = cc + ND
            if nxt < CPP:
                gather(pb, nxt, r, sems[r])
            elif ph + 1 < NPH:
                gather((ph + 1) & 1, nxt - CPP, r, sems[r])

    plsc.subcore_barrier()
    pltpu.sync_copy(agg_sh.at[pl.ds(s * RPS, RPS)], out_hbm.at[c, s])


@functools.cache
def _sc_agg():
    return pl.kernel(
        _sc_agg_body,
        out_type=jax.ShapeDtypeStruct((NC, NS, RPS, D), jnp.float32),
        mesh=plsc.VectorSubcoreMesh(core_axis_name="c", subcore_axis_name="s",
                                    num_cores=NC, num_subcores=NS),
        scratch_types=[
            pltpu.VMEM((2, 2, CPP, CH), jnp.int32),
            pltpu.VMEM((ND, CH, D), jnp.float32),
            pltpu.SemaphoreType.DMA,
            pltpu.SemaphoreType.DMA,
            pltpu.SemaphoreType.DMA,
            pltpu.SemaphoreType.DMA,
            pltpu.VMEM_SHARED((NP, D), jnp.float32),
        ],
    )

NB = 5
BLK = N // NB


def _tc_body(x_ref, agg_ref, bat_ref, w1_ref, b1_ref, w2_ref, b2_ref,
             l1w_ref, l1b_ref, l2w_ref, l2b_ref, emb_ref, z_ref, acc_ref):
    i = pl.program_id(0)

    @pl.when(i == 0)
    def _init():
        acc_ref[...] = jnp.zeros_like(acc_ref)

    h = x_ref[...] + agg_ref[0] + agg_ref[1]
    gids = lax.broadcasted_iota(jnp.int32, (BLK, G), 1).astype(jnp.float32)
    onehot = (bat_ref[...] == gids).astype(jnp.float32)
    for l in range(L):
        t = jnp.dot(h, w1_ref[l], preferred_element_type=jnp.float32)
        t = jnp.maximum(t + b1_ref[l, 0], 0.0)
        e = jnp.dot(t, w2_ref[l], preferred_element_type=jnp.float32)
        e = jnp.maximum(e + b2_ref[l, 0], 0.0)
        if l == L - 1:
            emb_ref[...] = e
        pooled = lax.dot_general(onehot, e, (((0,), (0,)), ((), ())),
                                 preferred_element_type=jnp.float32)
        acc_ref[:, l * H:(l + 1) * H] += pooled

    @pl.when(i == NB - 1)
    def _head():
        hc = acc_ref[...]
        zz = jnp.dot(hc, l1w_ref[...], preferred_element_type=jnp.float32)
        zz = jnp.maximum(zz + l1b_ref[0], 0.0)
        lg = jnp.dot(zz, l2w_ref[...], preferred_element_type=jnp.float32)
        lg = lg + l2b_ref[0]
        mx = jnp.max(lg, axis=1, keepdims=True)
        lse = jnp.log(jnp.sum(jnp.exp(lg - mx), axis=1, keepdims=True)) + mx
        z_ref[...] = lg - lse


_tc_call = pl.pallas_call(
    _tc_body,
    grid=(NB,),
    in_specs=[
        pl.BlockSpec((BLK, D), lambda i: (i, 0)),
        pl.BlockSpec((NC, BLK, D), lambda i: (0, i, 0)),
        pl.BlockSpec((BLK, 1), lambda i: (i, 0)),
        pl.BlockSpec((L, D, H), lambda i: (0, 0, 0)),
        pl.BlockSpec((L, 8, H), lambda i: (0, 0, 0)),
        pl.BlockSpec((L, H, H), lambda i: (0, 0, 0)),
        pl.BlockSpec((L, 8, H), lambda i: (0, 0, 0)),
        pl.BlockSpec((L * H, L * H), lambda i: (0, 0)),
        pl.BlockSpec((8, L * H), lambda i: (0, 0)),
        pl.BlockSpec((L * H, C), lambda i: (0, 0)),
        pl.BlockSpec((8, C), lambda i: (0, 0)),
    ],
    out_specs=[
        pl.BlockSpec((BLK, D), lambda i: (i, 0)),
        pl.BlockSpec((G, C), lambda i: (0, 0)),
    ],
    out_shape=[
        jax.ShapeDtypeStruct((N, D), jnp.float32),
        jax.ShapeDtypeStruct((G, C), jnp.float32),
    ],
    scratch_shapes=[pltpu.VMEM((G, L * H), jnp.float32)],
    compiler_params=pltpu.CompilerParams(
        dimension_semantics=("arbitrary",)),
)


def _pad8(v2d):
    return jnp.zeros((8, v2d.shape[1]), v2d.dtype).at[0].set(v2d[0])


def kernel(x, edge_index, batch, params):
    ei = edge_index.reshape(2, NW, NPH, CPP, CH).transpose(1, 2, 0, 3, 4)
    zeros = jnp.zeros((8, D), jnp.float32)
    aggs = _sc_agg()(x, ei, zeros).reshape(NC, NP, D)

    w1s, b1s, w2s, b2s = [], [], [], []
    for p in params["layers"]:
        scale = p["bn_gamma"] * lax.rsqrt(p["bn_var"] + 1e-5)
        w1s.append(p["w1"] * scale[None, :])
        b1s.append((p["b1"] - p["bn_mean"]) * scale + p["bn_beta"])
        w2s.append(p["w2"])
        b2s.append(p["b2"])
    w1 = jnp.stack(w1s)
    b1 = jnp.stack([jnp.zeros((8, H), jnp.float32).at[0].set(b) for b in b1s])
    w2 = jnp.stack(w2s)
    b2 = jnp.stack([jnp.zeros((8, H), jnp.float32).at[0].set(b) for b in b2s])
    l1b = _pad8(params["lin1_b"][None, :])
    l2b = _pad8(params["lin2_b"][None, :])
    batf = batch.astype(jnp.float32).reshape(N, 1)

    emb, z = _tc_call(x, aggs, batf, w1, b1, w2, b2,
                      params["lin1_w"], l1b, params["lin2_w"], l2b)
    return (emb, z)


# speedup vs baseline: 2.8968x; 2.8968x over previous
import functools

import jax
import jax.numpy as jnp
from jax import lax
from jax.experimental import pallas as pl
from jax.experimental.pallas import tpu as pltpu
from jax.experimental.pallas import tpu_sc as plsc

N = 10000
E = 320000
D = 128
H = 128
L = 3
C = 10
G = 64

NC = 2
NS = 16
NW = NC * NS
EPW = E // NW
CH = 40
CB = EPW // CH
NPH = 5
CPP = CB // NPH
ND = 4
NP = 10240
RPS = NP // NS


def _sc_agg_body(x_hbm, ei_hbm, zeros_hbm, out_hbm,
                 idx_v, rows_v, sem0, sem1, sem2, sem3, agg_sh):
    c = lax.axis_index("c")
    s = lax.axis_index("s")
    w = s * NC + c
    sems = (sem0, sem1, sem2, sem3)

    def load_phase(ph, parity):
        pltpu.sync_copy(ei_hbm.at[w].at[ph], idx_v.at[parity])

    def gather(parity, cc, buf, sm):
        pltpu.async_copy(x_hbm.at[idx_v.at[parity].at[0].at[cc]],
                         rows_v.at[buf], sm)

    def wait_one(sm):
        pltpu.make_async_copy(x_hbm.at[idx_v.at[0].at[0].at[0]],
                              rows_v.at[0], sm).wait()

    def scatter(parity, cc, buf):
        pltpu.sync_copy(rows_v.at[buf], agg_sh.at[idx_v.at[parity].at[1].at[cc]],
                        add=True)

    load_phase(0, 0)
    for j in range(ND):
        gather(0, j, j, sems[j])
    pltpu.sync_copy(zeros_hbm, agg_sh.at[pl.ds(s * RPS, RPS)])
    plsc.subcore_barrier()

    for ph in range(NPH):
        pb = ph & 1
        if ph + 1 < NPH:
            load_phase(ph + 1, (ph + 1) & 1)
        rs = [(2 * ph + j) % ND for j in range(ND)]

        def inner(v, carry, pb=pb, rs=rs):
            cc = ND * v
            for j in range(ND):
                r = rs[j]
                wait_one(sems[r])
                scatter(pb, cc + j, r)
                gather(pb, cc + j + ND, r, sems[r])
            return carry

        lax.fori_loop(0, (CPP - 2 * ND) // ND + 1, inner, 0)
        for cc in range(CPP - 2 - ND, CPP):
            r = (2 * ph + cc) % ND
            wait_one(sems[r])
            scatter(pb, cc, r)
            nxt = cc + ND
            if nxt < CPP:
                gather(pb, nxt, r, sems[r])
            elif ph + 1 < NPH:
                gather((ph + 1) & 1, nxt - CPP, r, sems[r])

    plsc.subcore_barrier()
    pltpu.sync_copy(agg_sh.at[pl.ds(s * RPS, RPS)], out_hbm.at[c, s])


@functools.cache
def _sc_agg():
    return pl.kernel(
        _sc_agg_body,
        out_type=jax.ShapeDtypeStruct((NC, NS, RPS, D), jnp.float32),
        mesh=plsc.VectorSubcoreMesh(core_axis_name="c", subcore_axis_name="s",
                                    num_cores=NC, num_subcores=NS),
        scratch_types=[
            pltpu.VMEM((2, 2, CPP, CH), jnp.int32),
            pltpu.VMEM((ND, CH, D), jnp.float32),
            pltpu.SemaphoreType.DMA,
            pltpu.SemaphoreType.DMA,
            pltpu.SemaphoreType.DMA,
            pltpu.SemaphoreType.DMA,
            pltpu.VMEM_SHARED((NP, D), jnp.float32),
        ],
    )

NB = 5
BLK = N // NB


def _tc_body(x_ref, agg_ref, bat_ref, w1_ref, b1_ref, w2_ref, b2_ref,
             l1w_ref, l1b_ref, l2w_ref, l2b_ref, emb_ref, z_ref, acc_ref):
    i = pl.program_id(0)

    @pl.when(i == 0)
    def _init():
        acc_ref[...] = jnp.zeros_like(acc_ref)

    h = x_ref[...] + agg_ref[0] + agg_ref[1]
    gids = lax.broadcasted_iota(jnp.int32, (BLK, G), 1).astype(jnp.float32)
    onehot = (bat_ref[...] == gids).astype(jnp.float32)
    for l in range(L):
        t = jnp.dot(h, w1_ref[l], preferred_element_type=jnp.float32)
        t = jnp.maximum(t + b1_ref[l, 0], 0.0)
        e = jnp.dot(t, w2_ref[l], preferred_element_type=jnp.float32)
        e = jnp.maximum(e + b2_ref[l, 0], 0.0)
        if l == L - 1:
            emb_ref[...] = e
        pooled = lax.dot_general(onehot, e, (((0,), (0,)), ((), ())),
                                 preferred_element_type=jnp.float32)
        acc_ref[:, l * H:(l + 1) * H] += pooled

    @pl.when(i == NB - 1)
    def _head():
        hc = acc_ref[...]
        zz = jnp.dot(hc, l1w_ref[...], preferred_element_type=jnp.float32)
        zz = jnp.maximum(zz + l1b_ref[0], 0.0)
        lg = jnp.dot(zz, l2w_ref[...], preferred_element_type=jnp.float32)
        lg = lg + l2b_ref[0]
        mx = jnp.max(lg, axis=1, keepdims=True)
        lse = jnp.log(jnp.sum(jnp.exp(lg - mx), axis=1, keepdims=True)) + mx
        z_ref[...] = lg - lse


_tc_call = pl.pallas_call(
    _tc_body,
    grid=(NB,),
    in_specs=[
        pl.BlockSpec((BLK, D), lambda i: (i, 0)),
        pl.BlockSpec((NC, BLK, D), lambda i: (0, i, 0)),
        pl.BlockSpec((BLK, 1), lambda i: (i, 0)),
        pl.BlockSpec((L, D, H), lambda i: (0, 0, 0)),
        pl.BlockSpec((L, 8, H), lambda i: (0, 0, 0)),
        pl.BlockSpec((L, H, H), lambda i: (0, 0, 0)),
        pl.BlockSpec((L, 8, H), lambda i: (0, 0, 0)),
        pl.BlockSpec((L * H, L * H), lambda i: (0, 0)),
        pl.BlockSpec((8, L * H), lambda i: (0, 0)),
        pl.BlockSpec((L * H, C), lambda i: (0, 0)),
        pl.BlockSpec((8, C), lambda i: (0, 0)),
    ],
    out_specs=[
        pl.BlockSpec((BLK, D), lambda i: (i, 0)),
        pl.BlockSpec((G, C), lambda i: (0, 0)),
    ],
    out_shape=[
        jax.ShapeDtypeStruct((N, D), jnp.float32),
        jax.ShapeDtypeStruct((G, C), jnp.float32),
    ],
    scratch_shapes=[pltpu.VMEM((G, L * H), jnp.float32)],
    compiler_params=pltpu.CompilerParams(
        dimension_semantics=("arbitrary",)),
)


def _pad8(v2d):
    return jnp.zeros((8, v2d.shape[1]), v2d.dtype).at[0].set(v2d[0])


def kernel(x, edge_index, batch, params):
    ei = edge_index.reshape(2, NW, NPH, CPP, CH).transpose(1, 2, 0, 3, 4)
    zeros = jnp.zeros((RPS, D), jnp.float32)
    aggs = _sc_agg()(x, ei, zeros).reshape(NC, NP, D)

    w1s, b1s, w2s, b2s = [], [], [], []
    for p in params["layers"]:
        scale = p["bn_gamma"] * lax.rsqrt(p["bn_var"] + 1e-5)
        w1s.append(p["w1"] * scale[None, :])
        b1s.append((p["b1"] - p["bn_mean"]) * scale + p["bn_beta"])
        w2s.append(p["w2"])
        b2s.append(p["b2"])
    w1 = jnp.stack(w1s)
    b1 = jnp.stack([jnp.zeros((8, H), jnp.float32).at[0].set(b) for b in b1s])
    w2 = jnp.stack(w2s)
    b2 = jnp.stack([jnp.zeros((8, H), jnp.float32).at[0].set(b) for b in b2s])
    l1b = _pad8(params["lin1_b"][None, :])
    l2b = _pad8(params["lin2_b"][None, :])
    batf = batch.astype(jnp.float32).reshape(N, 1)

    emb, z = _tc_call(x, aggs, batf, w1, b1, w2, b2,
                      params["lin1_w"], l1b, params["lin2_w"], l2b)
    return (emb, z)
